# baseline (speedup 1.0000x reference)
import functools

import jax
import jax.numpy as jnp
from jax import lax
from jax.experimental import pallas as pl
from jax.experimental.pallas import tpu as pltpu, tpu_sc as plsc

VOCAB = 1000000
D = 64
B = 4096 * 200
NC, NS = 2, 16
NW = NC * NS
B_PER_W = B // NW
CHUNK = 128
NCHUNK = B_PER_W // CHUNK
R = 3
L = 16

_mesh = plsc.VectorSubcoreMesh(
    core_axis_name="c", subcore_axis_name="s", num_cores=NC, num_subcores=NS
)


@functools.partial(
    pl.kernel,
    out_type=jax.ShapeDtypeStruct((B, D), jnp.float32),
    mesh=_mesh,
    scratch_types=[
        pltpu.VMEM((B_PER_W,), jnp.int32),
        pltpu.VMEM((R, CHUNK), jnp.int32),
        pltpu.VMEM((R, CHUNK), jnp.int32),
        pltpu.VMEM((R, CHUNK, 2 * D), jnp.float32),
        pltpu.VMEM((R, CHUNK, D), jnp.float32),
        pltpu.SemaphoreType.DMA,
        pltpu.SemaphoreType.DMA,
        pltpu.SemaphoreType.DMA,
        pltpu.SemaphoreType.DMA,
        pltpu.SemaphoreType.DMA,
        pltpu.SemaphoreType.DMA,
    ],
)
def _emb_lookup(idx_hbm, table_hbm, out_hbm, xv, qbuf, obuf, pairs_v, rows_v,
                g0, g1, g2, s0, s1, s2):
    wid = lax.axis_index("s") * NC + lax.axis_index("c")
    base = wid * B_PER_W
    gsem = (g0, g1, g2)
    ssem = (s0, s1, s2)

    pltpu.sync_copy(idx_hbm.at[pl.ds(base, B_PER_W)], xv)

    def fire_gather(gi, r):
        @pl.loop(0, CHUNK // L)
        def _prep(j):
            v = xv[pl.ds(gi * CHUNK + j * L, L)]
            obuf[r, pl.ds(j * L, L)] = (v & 1) << 6
            qbuf[r, pl.ds(j * L, L)] = v >> 1

        pltpu.make_async_copy(
            table_hbm.at[qbuf.at[r]], pairs_v.at[r], gsem[r]
        ).start()

    def gather_wait(r):
        pltpu.make_async_copy(
            table_hbm.at[qbuf.at[r]], pairs_v.at[r], gsem[r]
        ).wait()

    def store_desc(gi, r):
        out_sl = out_hbm.at[pl.ds(base + gi * CHUNK, CHUNK)]
        return pltpu.make_async_copy(rows_v.at[r], out_sl, ssem[r])

    def extract(r):
        @pl.loop(0, CHUNK // L)
        def _blk(b):
            ovec = obuf[r, pl.ds(b * L, L)]
            for j in range(L):
                off = ovec[j]
                i = b * L + j
                for k in range(D // L):
                    rows_v[r, i, pl.ds(k * L, L)] = pairs_v[r, i, pl.ds(off + k * L, L)]

    fire_gather(0, 0)
    fire_gather(1, 1)
    gather_wait(0)
    extract(0)
    store_desc(0, 0).start()
    fire_gather(2, 2)
    gather_wait(1)
    extract(1)
    store_desc(1, 1).start()

    @pl.loop(2, NCHUNK - 3, step=3)
    def _steady(i):
        for p in range(3):
            g = i + p
            r = (2 + p) % R
            rn = (r + 1) % R
            store_desc(g - 2, rn).wait()
            fire_gather(g + 1, rn)
            gather_wait(r)
            extract(r)
            store_desc(g, r).start()

    store_desc(195, 0).wait()
    fire_gather(198, 0)
    gather_wait(2)
    extract(2)
    store_desc(197, 2).start()

    store_desc(196, 1).wait()
    fire_gather(199, 1)
    gather_wait(0)
    extract(0)
    store_desc(198, 0).start()

    store_desc(197, 2).wait()
    gather_wait(1)
    extract(1)
    store_desc(199, 1).start()

    store_desc(198, 0).wait()
    store_desc(199, 1).wait()


def kernel(x, table):
    table2 = table.reshape(VOCAB // 2, 2 * D)
    out = _emb_lookup(x.reshape(-1), table2)
    return out.reshape(x.shape + (D,))
